# baseline (speedup 1.0000x reference)
import functools

import jax
import jax.numpy as jnp
from jax import lax
from jax.experimental import pallas as pl
from jax.experimental.pallas import tpu as pltpu
from jax.experimental.pallas import tpu_sc as plsc

_EMB = 16
_HID = 128
_IN = 3217
_CHUNKS = 201
_ROWS_PER_TILE = 8

_mesh = plsc.VectorSubcoreMesh(
    core_axis_name="c", subcore_axis_name="s", num_cores=1
)


_SC_CFG = dict(
    out_type=jax.ShapeDtypeStruct((16,), jnp.float32),
    mesh=_mesh,
    compiler_params=pltpu.CompilerParams(
        needs_layout_passes=False, use_tc_tiling_on_sc=False
    ),
    scratch_types=[
        pltpu.VMEM((128,), jnp.int32),
        pltpu.VMEM((128,), jnp.int32),
        pltpu.VMEM((16,), jnp.int32),
        pltpu.VMEM((264, _EMB), jnp.float32),
        pltpu.VMEM((16, _EMB), jnp.float32),
        pltpu.VMEM((_ROWS_PER_TILE, _IN), jnp.float32),
        pltpu.VMEM((16, 16), jnp.float32),
        pltpu.VMEM((16, 16), jnp.float32),
        pltpu.VMEM((16,), jnp.float32),
        pltpu.VMEM((16, 16), jnp.float32),
        pltpu.VMEM((16,), jnp.float32),
        pltpu.VMEM_SHARED((16, 16), jnp.float32),
        pltpu.SemaphoreType.DMA,
        pltpu.SemaphoreType.DMA,
        pltpu.SemaphoreType.DMA,
    ],
)


def _sc_body(word_ref, spk_ref, w1_ref, b1_ref, w2_ref, idx_ref, sp_ref,
                aux_ref, out_ref, idx_a, idx_b, sp_idx, xrows, sp_rows, w1_v,
                b1_v, w2_v, aux_v, psum_v, res_v, part_sh, sem0, sem1, sem2):
    s = lax.axis_index("s")

    pltpu.sync_copy(idx_ref.at[0], idx_a)
    pltpu.sync_copy(idx_ref.at[1], idx_b)
    pltpu.sync_copy(sp_ref, sp_idx)
    pltpu.sync_copy(aux_ref, aux_v)
    pltpu.sync_copy(b1_ref, b1_v)
    pltpu.sync_copy(w2_ref, w2_v)

    cp0 = pltpu.async_copy(word_ref.at[idx_a], xrows.at[pl.ds(1, 128)], sem0)
    cp1 = pltpu.async_copy(word_ref.at[idx_b], xrows.at[pl.ds(129, 128)], sem1)
    cp2 = pltpu.async_copy(spk_ref.at[sp_idx], sp_rows, sem2)

    pltpu.sync_copy(w1_ref.at[pl.ds(s * _ROWS_PER_TILE, _ROWS_PER_TILE)], w1_v)
    cp0.wait()
    cp1.wait()
    cp2.wait()
    xrows[0] = sp_rows[0]

    def dot_body(j, accs):
        xj = xrows[j]
        return tuple(
            accs[r] + w1_v[r, pl.ds(j * 16, 16)] * xj
            for r in range(_ROWS_PER_TILE)
        )

    zero = jnp.zeros((16,), jnp.float32)
    accs = lax.fori_loop(
        0, _CHUNKS, dot_body, tuple(zero for _ in range(_ROWS_PER_TILE))
    )

    aux_vec = aux_v[...]
    quant = aux_vec[0]
    bvec = b1_v[s]
    w2vec = w2_v[s]
    partial = jnp.float32(0.0)
    for r in range(_ROWS_PER_TILE):
        wlast = w1_v[r, pl.ds(_IN - 16, 16)]
        h = jnp.sum(accs[r]) + quant * wlast[15] + bvec[r]
        h = jnp.maximum(h, 0.0)
        partial = partial + h * w2vec[r]

    res_v[...] = jnp.full((16,), partial, jnp.float32)
    pltpu.sync_copy(res_v, part_sh.at[s])
    plsc.subcore_barrier()

    @pl.when(s == 0)
    def _():
        pltpu.sync_copy(part_sh, psum_v)
        tot = psum_v[0]
        for i in range(1, 16):
            tot = tot + psum_v[i]
        z = tot + aux_v[...][1]
        res_v[...] = 1.0 / (1.0 + jnp.exp(-z))
        pltpu.sync_copy(res_v, out_ref)


_sc_forward = pl.kernel(_sc_body, **_SC_CFG)


def kernel(context_indices, speaker, col_three_indices, quant, sentiment,
           word_emb, speaker_emb, W1, b1, W2, b2):
    del sentiment
    ctx = context_indices.astype(jnp.int32)
    c3 = col_three_indices.astype(jnp.int32)
    idx2d = jnp.concatenate([ctx, jnp.broadcast_to(c3, (57,))]).reshape(2, 128)
    sp16 = jnp.broadcast_to(speaker.astype(jnp.int32), (16,))
    aux = jnp.concatenate(
        [quant.astype(jnp.float32), b2.astype(jnp.float32),
         jnp.zeros((14,), jnp.float32)]
    )
    b1p = jnp.pad(b1.reshape(16, 8), ((0, 0), (0, 8)))
    w2p = jnp.pad(W2.reshape(16, 8), ((0, 0), (0, 8)))
    out16 = _sc_forward(word_emb, speaker_emb, W1, b1p, w2p, idx2d, sp16, aux)
    return out16[:1].reshape(1, 1)
